# Initial kernel scaffold; baseline (speedup 1.0000x reference)
import jax
import jax.numpy as jnp
from jax.experimental import pallas as pl


def kernel(p, Q, K, V, b, W1, b1, gamma, beta, rm, rv, W2, b2):
    raise NotImplementedError("write your pallas kernel here")


# speedup vs baseline: 8.9038x; 8.9038x over previous
import functools

import jax
import jax.numpy as jnp
from jax import lax
from jax.experimental import pallas as pl
from jax.experimental.pallas import tpu as pltpu
from jax.experimental.pallas import tpu_sc as plsc

_HEADS = 8
_DIM = 256
_KNN = 16
_SCALE = 5.656854249
_NB = 4
_DH = _DIM // _HEADS

_R = 128
_C = 2048
_NC = 2
_NS = 16
_NW = _NC * _NS
_CH = 128


def _knn_body(lo_ref, hi_ref, pqx_ref, pqy_ref, pqz_ref, bq_ref,
              pkx_ref, pky_ref, pkz_ref, bk_ref, out_ref):
    i = pl.program_id(0)
    lo = lo_ref[i]
    hi = hi_ref[i]
    qx = pqx_ref[:, :]
    qy = pqy_ref[:, :]
    qz = pqz_ref[:, :]
    bq = bq_ref[:, :]
    r = qx.shape[0]
    cw = _C + _KNN
    lane = lax.broadcasted_iota(jnp.int32, (r, cw), 1)
    lane_c = lax.broadcasted_iota(jnp.int32, (r, _C), 1)
    lane_k = lax.broadcasted_iota(jnp.int32, (r, _KNN), 1)
    inf = jnp.float32(jnp.inf)

    def chunk(t, carry):
        vals, gidx = carry
        kx = pkx_ref[pl.ds(t, 1), :]
        ky = pky_ref[pl.ds(t, 1), :]
        kz = pkz_ref[pl.ds(t, 1), :]
        bk = bk_ref[pl.ds(t, 1), :]
        d = (qx - kx) ** 2 + (qy - ky) ** 2 + (qz - kz) ** 2
        d = jnp.where(bq == bk, d, inf)
        work = jnp.concatenate([d, vals], axis=1)
        s = t * _C
        new_vals = []
        new_idx = []
        for _ in range(_KNN):
            mn = jnp.min(work, axis=1, keepdims=True)
            col = jnp.min(jnp.where(work <= mn, lane, cw), axis=1, keepdims=True)
            carry_pick = jnp.sum(
                jnp.where(lane_k == col - _C, gidx, 0), axis=1, keepdims=True)
            pick = jnp.where(col < _C, s + col, carry_pick)
            new_vals.append(mn)
            new_idx.append(pick)
            work = jnp.where(lane == col, inf, work)
        return jnp.concatenate(new_vals, axis=1), jnp.concatenate(new_idx, axis=1)

    vals0 = jnp.full((r, _KNN), inf, dtype=jnp.float32)
    gidx0 = jnp.zeros((r, _KNN), dtype=jnp.int32)
    _, gidx = lax.fori_loop(lo // _C, (hi + _C - 1) // _C, chunk, (vals0, gidx0))
    out_ref[:, :] = gidx


def _attn_body(q_ref, kvp_ref, pq_ref,
               w1_ref, b1_ref, w2_ref, b2_ref, out_ref):
    f32 = jnp.float32
    r = q_ref.shape[0]
    g = r * _KNN
    q = q_ref[:, :]
    kg = kvp_ref[:, :_DIM]
    vg = kvp_ref[:, _DIM:2 * _DIM]
    pg = kvp_ref[:, 2 * _DIM:2 * _DIM + 16]
    pq = pq_ref[:, :]
    q_rep = jnp.broadcast_to(q[:, None, :], (r, _KNN, _DIM)).reshape(g, _DIM)
    pq_rep = jnp.broadcast_to(pq[:, None, :], (r, _KNN, 16)).reshape(g, 16)
    dp = pq_rep - pg
    h = jnp.dot(dp, w1_ref[:, :], preferred_element_type=f32) + b1_ref[:, :]
    h = jnp.maximum(h, 0.0)
    pe = jnp.dot(h, w2_ref[:, :], preferred_element_type=f32) + b2_ref[:, :]
    s_pe = jnp.sum(pe, axis=1, keepdims=True)
    hsel = (lax.broadcasted_iota(jnp.int32, (_DIM, _HEADS), 0) // _DH
            == lax.broadcasted_iota(jnp.int32, (_DIM, _HEADS), 1)).astype(f32)
    qk = jnp.dot(kg * q_rep, hsel, preferred_element_type=f32)
    w = (qk + s_pe) * (1.0 / (_SCALE * _KNN + 1e-8))
    w3 = w.reshape(r, _KNN, _HEADS)
    m = jnp.max(w3, axis=1, keepdims=True)
    e = jnp.exp(w3 - m)
    a3 = e / jnp.sum(e, axis=1, keepdims=True)
    a = a3.reshape(g, _HEADS)
    hexp = (lax.broadcasted_iota(jnp.int32, (_HEADS, _DIM), 0)
            == lax.broadcasted_iota(jnp.int32, (_HEADS, _DIM), 1) // _DH).astype(f32)
    a_exp = jnp.dot(a, hexp, preferred_element_type=f32)
    texp = (lax.broadcasted_iota(jnp.int32, (_DH, _DIM), 0)
            == lax.broadcasted_iota(jnp.int32, (_DH, _DIM), 1) % _DH).astype(f32)
    pe_exp = jnp.dot(pe, texp, preferred_element_type=f32)
    contrib = (vg + pe_exp) * a_exp
    out_ref[:, :] = jnp.sum(contrib.reshape(r, _KNN, _DIM), axis=1)


_KVP = 2 * _DIM + 128


def _gather_body(idx_hbm, kvp_hbm, out_hbm, idx_v, buf, sem, *, total):
    c = lax.axis_index("c")
    s = lax.axis_index("s")
    wid = s * _NC + c
    per_w = total // _NW
    base = wid * per_w

    def chunk(t, carry):
        row0 = base + t * _CH
        pltpu.sync_copy(idx_hbm.at[pl.ds(row0, _CH)], idx_v)
        pltpu.async_copy(kvp_hbm.at[idx_v], buf, sem).wait()
        pltpu.sync_copy(buf, out_hbm.at[pl.ds(row0, _CH)])
        return carry

    lax.fori_loop(0, per_w // _CH, chunk, 0)


def _knn_call(n, lo_blk, hi_blk, pqx, pqy, pqz, bq2, pkx, pky, pkz, bkr):
    nblk = n // _R
    nrow = pkx.shape[0]
    grid_spec = pltpu.PrefetchScalarGridSpec(
        num_scalar_prefetch=2,
        grid=(nblk,),
        in_specs=[
            pl.BlockSpec((_R, 1), lambda i, lo, hi: (i, 0)),
            pl.BlockSpec((_R, 1), lambda i, lo, hi: (i, 0)),
            pl.BlockSpec((_R, 1), lambda i, lo, hi: (i, 0)),
            pl.BlockSpec((_R, 1), lambda i, lo, hi: (i, 0)),
            pl.BlockSpec((nrow, _C), lambda i, lo, hi: (0, 0)),
            pl.BlockSpec((nrow, _C), lambda i, lo, hi: (0, 0)),
            pl.BlockSpec((nrow, _C), lambda i, lo, hi: (0, 0)),
            pl.BlockSpec((nrow, _C), lambda i, lo, hi: (0, 0)),
        ],
        out_specs=pl.BlockSpec((_R, _KNN), lambda i, lo, hi: (i, 0)),
    )
    return pl.pallas_call(
        _knn_body,
        grid_spec=grid_spec,
        out_shape=jax.ShapeDtypeStruct((n, _KNN), jnp.int32),
    )(lo_blk, hi_blk, pqx, pqy, pqz, bq2, pkx, pky, pkz, bkr)


def _attn_call(n, Q, kvpg, ppad, w1pad, b1pad, w2pad, b2row):
    nblk = n // _R
    return pl.pallas_call(
        _attn_body,
        grid=(nblk,),
        in_specs=[
            pl.BlockSpec((_R, _DIM), lambda i: (i, 0)),
            pl.BlockSpec((_R * _KNN, _KVP), lambda i: (i, 0)),
            pl.BlockSpec((_R, 16), lambda i: (i, 0)),
            pl.BlockSpec((16, 16), lambda i: (0, 0)),
            pl.BlockSpec((1, 16), lambda i: (0, 0)),
            pl.BlockSpec((16, _DH), lambda i: (0, 0)),
            pl.BlockSpec((1, _DH), lambda i: (0, 0)),
        ],
        out_specs=pl.BlockSpec((_R, _DIM), lambda i: (i, 0)),
        out_shape=jax.ShapeDtypeStruct((n, _DIM), jnp.float32),
    )(Q, kvpg, ppad, w1pad, b1pad, w2pad, b2row)


def _gather_call(n, idx_flat, kvp):
    total = n * _KNN
    mesh = plsc.VectorSubcoreMesh(core_axis_name="c", subcore_axis_name="s")
    body = functools.partial(_gather_body, total=total)
    return pl.kernel(
        body,
        out_type=jax.ShapeDtypeStruct((total, _KVP), jnp.float32),
        mesh=mesh,
        scratch_types=[
            pltpu.VMEM((_CH,), jnp.int32),
            pltpu.VMEM((_CH, _KVP), jnp.float32),
            pltpu.SemaphoreType.DMA,
        ],
    )(idx_flat, kvp)


def kernel(p, Q, K, V, b, W1, b1, gamma, beta, rm, rv, W2, b2):
    n = Q.shape[0]
    f32 = jnp.float32
    sc = gamma / jnp.sqrt(rv + 1e-5)
    w1f = W1 * sc[None, :]
    b1f = (b1 - rm) * sc + beta
    w1pad = jnp.zeros((16, 16), f32).at[:3, :3].set(w1f)
    b1pad = jnp.zeros((1, 16), f32).at[0, :3].set(b1f)
    w2pad = jnp.zeros((16, _DH), f32).at[:3, :].set(W2)
    b2row = b2.reshape(1, _DH).astype(f32)

    offs = jnp.searchsorted(b, jnp.arange(_NB + 1, dtype=b.dtype)).astype(jnp.int32)
    first = b[::_R]
    last = b[_R - 1::_R]
    lo_blk = offs[first]
    hi_blk = offs[last + 1]

    npad = ((n + _C - 1) // _C) * _C
    padn = npad - n
    px = jnp.pad(p[:, 0], (0, padn)).reshape(npad // _C, _C)
    py = jnp.pad(p[:, 1], (0, padn)).reshape(npad // _C, _C)
    pz = jnp.pad(p[:, 2], (0, padn)).reshape(npad // _C, _C)
    bkr = jnp.pad(b, (0, padn), constant_values=-1).reshape(npad // _C, _C)
    pqx = p[:, 0].reshape(n, 1)
    pqy = p[:, 1].reshape(n, 1)
    pqz = p[:, 2].reshape(n, 1)
    bq2 = b.reshape(n, 1)

    idx = _knn_call(n, lo_blk, hi_blk, pqx, pqy, pqz, bq2, px, py, pz, bkr)

    ppad = jnp.zeros((n, 16), f32).at[:, :3].set(p)
    kvp = jnp.concatenate(
        [K, V, ppad, jnp.zeros((n, _KVP - 2 * _DIM - 16), f32)], axis=1)
    kvpg = _gather_call(n, idx.reshape(n * _KNN), kvp)
    return _attn_call(n, Q, kvpg, ppad, w1pad, b1pad, w2pad, b2row)


# speedup vs baseline: 10.1413x; 1.1390x over previous
import functools

import jax
import jax.numpy as jnp
from jax import lax
from jax.experimental import pallas as pl
from jax.experimental.pallas import tpu as pltpu
from jax.experimental.pallas import tpu_sc as plsc

_HEADS = 8
_DIM = 256
_KNN = 16
_SCALE = 5.656854249
_NB = 4
_DH = _DIM // _HEADS

_R = 128
_C = 1024
_NC = 2
_NS = 16
_NW = _NC * _NS
_CH = 128


def _knn_body(lo_ref, hi_ref, pqx_ref, pqy_ref, pqz_ref, bq_ref,
              pkx_ref, pky_ref, pkz_ref, bk_ref, out_ref):
    i = pl.program_id(0)
    lo = lo_ref[i]
    hi = hi_ref[i]
    qx = pqx_ref[:, :]
    qy = pqy_ref[:, :]
    qz = pqz_ref[:, :]
    bq = bq_ref[:, :]
    r = qx.shape[0]
    cw = _C + _KNN
    lane = lax.broadcasted_iota(jnp.int32, (r, cw), 1)
    lane_c = lax.broadcasted_iota(jnp.int32, (r, _C), 1)
    lane_k = lax.broadcasted_iota(jnp.int32, (r, _KNN), 1)
    inf = jnp.float32(jnp.inf)

    def chunk(t, carry):
        vals, gidx = carry
        kx = pkx_ref[pl.ds(t, 1), :]
        ky = pky_ref[pl.ds(t, 1), :]
        kz = pkz_ref[pl.ds(t, 1), :]
        bk = bk_ref[pl.ds(t, 1), :]
        d = (qx - kx) ** 2 + (qy - ky) ** 2 + (qz - kz) ** 2
        d = jnp.where(bq == bk, d, inf)
        work = jnp.concatenate([d, vals], axis=1)
        s = t * _C
        new_vals = []
        new_idx = []
        for _ in range(_KNN):
            mn = jnp.min(work, axis=1, keepdims=True)
            le = work <= mn
            col = jnp.min(jnp.where(le, lane, cw), axis=1, keepdims=True)
            carry_pick = jnp.sum(
                jnp.where(lane_k == col - _C, gidx, 0), axis=1, keepdims=True)
            pick = jnp.where(col < _C, s + col, carry_pick)
            new_vals.append(mn)
            new_idx.append(pick)
            work = jnp.where(le, inf, work)
        return jnp.concatenate(new_vals, axis=1), jnp.concatenate(new_idx, axis=1)

    vals0 = jnp.full((r, _KNN), inf, dtype=jnp.float32)
    gidx0 = jnp.zeros((r, _KNN), dtype=jnp.int32)
    _, gidx = lax.fori_loop(lo // _C, (hi + _C - 1) // _C, chunk, (vals0, gidx0))
    out_ref[:, :] = gidx


def _attn_body(q_ref, kvp_ref, pq_ref,
               w1_ref, b1_ref, w2_ref, b2_ref, out_ref):
    f32 = jnp.float32
    r = q_ref.shape[0]
    g = r * _KNN
    q = q_ref[:, :]
    kg = kvp_ref[:, :_DIM]
    vg = kvp_ref[:, _DIM:2 * _DIM]
    pg = kvp_ref[:, 2 * _DIM:2 * _DIM + 16]
    pq = pq_ref[:, :]
    q_rep = jnp.broadcast_to(q[:, None, :], (r, _KNN, _DIM)).reshape(g, _DIM)
    pq_rep = jnp.broadcast_to(pq[:, None, :], (r, _KNN, 16)).reshape(g, 16)
    dp = pq_rep - pg
    h = jnp.dot(dp, w1_ref[:, :], preferred_element_type=f32) + b1_ref[:, :]
    h = jnp.maximum(h, 0.0)
    pe = jnp.dot(h, w2_ref[:, :], preferred_element_type=f32) + b2_ref[:, :]
    s_pe = jnp.sum(pe, axis=1, keepdims=True)
    hsel = (lax.broadcasted_iota(jnp.int32, (_DIM, _HEADS), 0) // _DH
            == lax.broadcasted_iota(jnp.int32, (_DIM, _HEADS), 1)).astype(f32)
    qk = jnp.dot(kg * q_rep, hsel, preferred_element_type=f32)
    w = (qk + s_pe) * (1.0 / (_SCALE * _KNN + 1e-8))
    w3 = w.reshape(r, _KNN, _HEADS)
    m = jnp.max(w3, axis=1, keepdims=True)
    e = jnp.exp(w3 - m)
    a3 = e / jnp.sum(e, axis=1, keepdims=True)
    a = a3.reshape(g, _HEADS)
    hexp = (lax.broadcasted_iota(jnp.int32, (_HEADS, _DIM), 0)
            == lax.broadcasted_iota(jnp.int32, (_HEADS, _DIM), 1) // _DH).astype(f32)
    a_exp = jnp.dot(a, hexp, preferred_element_type=f32)
    texp = (lax.broadcasted_iota(jnp.int32, (_DH, _DIM), 0)
            == lax.broadcasted_iota(jnp.int32, (_DH, _DIM), 1) % _DH).astype(f32)
    pe_exp = jnp.dot(pe, texp, preferred_element_type=f32)
    contrib = (vg + pe_exp) * a_exp
    out_ref[:, :] = jnp.sum(contrib.reshape(r, _KNN, _DIM), axis=1)


_KVP = 2 * _DIM + 128


def _gather_body(idx_hbm, kvp_hbm, out_hbm, idx_v, buf, sem, *, total):
    c = lax.axis_index("c")
    s = lax.axis_index("s")
    wid = s * _NC + c
    per_w = total // _NW
    base = wid * per_w

    def chunk(t, carry):
        row0 = base + t * _CH
        pltpu.sync_copy(idx_hbm.at[pl.ds(row0, _CH)], idx_v)
        pltpu.async_copy(kvp_hbm.at[idx_v], buf, sem).wait()
        pltpu.sync_copy(buf, out_hbm.at[pl.ds(row0, _CH)])
        return carry

    lax.fori_loop(0, per_w // _CH, chunk, 0)


def _knn_call(n, lo_blk, hi_blk, pqx, pqy, pqz, bq2, pkx, pky, pkz, bkr):
    nblk = n // _R
    nrow = pkx.shape[0]
    grid_spec = pltpu.PrefetchScalarGridSpec(
        num_scalar_prefetch=2,
        grid=(nblk,),
        in_specs=[
            pl.BlockSpec((_R, 1), lambda i, lo, hi: (i, 0)),
            pl.BlockSpec((_R, 1), lambda i, lo, hi: (i, 0)),
            pl.BlockSpec((_R, 1), lambda i, lo, hi: (i, 0)),
            pl.BlockSpec((_R, 1), lambda i, lo, hi: (i, 0)),
            pl.BlockSpec((nrow, _C), lambda i, lo, hi: (0, 0)),
            pl.BlockSpec((nrow, _C), lambda i, lo, hi: (0, 0)),
            pl.BlockSpec((nrow, _C), lambda i, lo, hi: (0, 0)),
            pl.BlockSpec((nrow, _C), lambda i, lo, hi: (0, 0)),
        ],
        out_specs=pl.BlockSpec((_R, _KNN), lambda i, lo, hi: (i, 0)),
    )
    return pl.pallas_call(
        _knn_body,
        grid_spec=grid_spec,
        out_shape=jax.ShapeDtypeStruct((n, _KNN), jnp.int32),
    )(lo_blk, hi_blk, pqx, pqy, pqz, bq2, pkx, pky, pkz, bkr)


def _attn_call(n, Q, kvpg, ppad, w1pad, b1pad, w2pad, b2row):
    nblk = n // _R
    return pl.pallas_call(
        _attn_body,
        grid=(nblk,),
        in_specs=[
            pl.BlockSpec((_R, _DIM), lambda i: (i, 0)),
            pl.BlockSpec((_R * _KNN, _KVP), lambda i: (i, 0)),
            pl.BlockSpec((_R, 16), lambda i: (i, 0)),
            pl.BlockSpec((16, 16), lambda i: (0, 0)),
            pl.BlockSpec((1, 16), lambda i: (0, 0)),
            pl.BlockSpec((16, _DH), lambda i: (0, 0)),
            pl.BlockSpec((1, _DH), lambda i: (0, 0)),
        ],
        out_specs=pl.BlockSpec((_R, _DIM), lambda i: (i, 0)),
        out_shape=jax.ShapeDtypeStruct((n, _DIM), jnp.float32),
    )(Q, kvpg, ppad, w1pad, b1pad, w2pad, b2row)


def _gather_call(n, idx_flat, kvp):
    total = n * _KNN
    mesh = plsc.VectorSubcoreMesh(core_axis_name="c", subcore_axis_name="s")
    body = functools.partial(_gather_body, total=total)
    return pl.kernel(
        body,
        out_type=jax.ShapeDtypeStruct((total, _KVP), jnp.float32),
        mesh=mesh,
        scratch_types=[
            pltpu.VMEM((_CH,), jnp.int32),
            pltpu.VMEM((_CH, _KVP), jnp.float32),
            pltpu.SemaphoreType.DMA,
        ],
    )(idx_flat, kvp)


def kernel(p, Q, K, V, b, W1, b1, gamma, beta, rm, rv, W2, b2):
    n = Q.shape[0]
    f32 = jnp.float32
    sc = gamma / jnp.sqrt(rv + 1e-5)
    w1f = W1 * sc[None, :]
    b1f = (b1 - rm) * sc + beta
    w1pad = jnp.zeros((16, 16), f32).at[:3, :3].set(w1f)
    b1pad = jnp.zeros((1, 16), f32).at[0, :3].set(b1f)
    w2pad = jnp.zeros((16, _DH), f32).at[:3, :].set(W2)
    b2row = b2.reshape(1, _DH).astype(f32)

    offs = jnp.searchsorted(b, jnp.arange(_NB + 1, dtype=b.dtype)).astype(jnp.int32)
    first = b[::_R]
    last = b[_R - 1::_R]
    lo_blk = offs[first]
    hi_blk = offs[last + 1]

    npad = ((n + _C - 1) // _C) * _C
    padn = npad - n
    px = jnp.pad(p[:, 0], (0, padn)).reshape(npad // _C, _C)
    py = jnp.pad(p[:, 1], (0, padn)).reshape(npad // _C, _C)
    pz = jnp.pad(p[:, 2], (0, padn)).reshape(npad // _C, _C)
    bkr = jnp.pad(b, (0, padn), constant_values=-1).reshape(npad // _C, _C)
    pqx = p[:, 0].reshape(n, 1)
    pqy = p[:, 1].reshape(n, 1)
    pqz = p[:, 2].reshape(n, 1)
    bq2 = b.reshape(n, 1)

    idx = _knn_call(n, lo_blk, hi_blk, pqx, pqy, pqz, bq2, px, py, pz, bkr)

    ppad = jnp.zeros((n, 16), f32).at[:, :3].set(p)
    kvp = jnp.concatenate(
        [K, V, ppad, jnp.zeros((n, _KVP - 2 * _DIM - 16), f32)], axis=1)
    kvpg = _gather_call(n, idx.reshape(n * _KNN), kvp)
    return _attn_call(n, Q, kvpg, ppad, w1pad, b1pad, w2pad, b2row)


# speedup vs baseline: 11.6631x; 1.1501x over previous
import functools

import jax
import jax.numpy as jnp
from jax import lax
from jax.experimental import pallas as pl
from jax.experimental.pallas import tpu as pltpu
from jax.experimental.pallas import tpu_sc as plsc

_HEADS = 8
_DIM = 256
_KNN = 16
_SCALE = 5.656854249
_NB = 4
_DH = _DIM // _HEADS

_R = 128
_C = 1024
_NC = 2
_NS = 16
_NW = _NC * _NS
_CH = 128


def _knn_body(lo_ref, hi_ref, pqx_ref, pqy_ref, pqz_ref, bq_ref,
              pkx_ref, pky_ref, pkz_ref, bk_ref, out_ref):
    i = pl.program_id(0)
    lo = lo_ref[i]
    hi = hi_ref[i]
    qx = pqx_ref[:, :]
    qy = pqy_ref[:, :]
    qz = pqz_ref[:, :]
    bq = bq_ref[:, :]
    r = qx.shape[0]
    cw = _C + _KNN
    lane = lax.broadcasted_iota(jnp.int32, (r, cw), 1)
    lane_k = lax.broadcasted_iota(jnp.int32, (r, _KNN), 1)
    inf = jnp.float32(jnp.inf)

    def chunk(t, carry):
        vals, gidx = carry
        kx = pkx_ref[pl.ds(t, 1), :]
        ky = pky_ref[pl.ds(t, 1), :]
        kz = pkz_ref[pl.ds(t, 1), :]
        bk = bk_ref[pl.ds(t, 1), :]
        d = (qx - kx) ** 2 + (qy - ky) ** 2 + (qz - kz) ** 2
        d = jnp.where(bq == bk, d, inf)
        work = jnp.concatenate([d, vals], axis=1)
        s = t * _C
        new_vals = []
        new_idx = []
        for _ in range(_KNN):
            mn = jnp.min(work, axis=1, keepdims=True)
            le = work <= mn
            col = jnp.min(jnp.where(le, lane, cw), axis=1, keepdims=True)
            carry_pick = jnp.sum(
                jnp.where(lane_k == col - _C, gidx, 0), axis=1, keepdims=True)
            pick = jnp.where(col < _C, s + col, carry_pick)
            new_vals.append(mn)
            new_idx.append(pick)
            work = jnp.where(le, inf, work)
        return jnp.concatenate(new_vals, axis=1), jnp.concatenate(new_idx, axis=1)

    vals0 = jnp.full((r, _KNN), inf, dtype=jnp.float32)
    gidx0 = jnp.zeros((r, _KNN), dtype=jnp.int32)
    _, gidx = lax.fori_loop(lo // _C, (hi + _C - 1) // _C, chunk, (vals0, gidx0))
    out_ref[:, :] = gidx


def _attn_body(q_ref, kvp_ref, pq_ref,
               w1_ref, b1_ref, w2_ref, b2_ref, out_ref):
    f32 = jnp.float32
    r = q_ref.shape[0]
    g = r * _KNN
    q = q_ref[:, :]
    kg = kvp_ref[:, :_DIM]
    vg = kvp_ref[:, _DIM:2 * _DIM]
    pg = kvp_ref[:, 2 * _DIM:2 * _DIM + 16]
    pq = pq_ref[:, :]
    q_rep = jnp.broadcast_to(q[:, None, :], (r, _KNN, _DIM)).reshape(g, _DIM)
    pq_rep = jnp.broadcast_to(pq[:, None, :], (r, _KNN, 16)).reshape(g, 16)
    dp = pq_rep - pg
    h = jnp.dot(dp, w1_ref[:, :], preferred_element_type=f32) + b1_ref[:, :]
    h = jnp.maximum(h, 0.0)
    pe = jnp.dot(h, w2_ref[:, :], preferred_element_type=f32) + b2_ref[:, :]
    s_pe = jnp.sum(pe, axis=1, keepdims=True)
    hsel = (lax.broadcasted_iota(jnp.int32, (_DIM, _HEADS), 0) // _DH
            == lax.broadcasted_iota(jnp.int32, (_DIM, _HEADS), 1)).astype(f32)
    qk = jnp.dot(kg * q_rep, hsel, preferred_element_type=f32)
    w = (qk + s_pe) * (1.0 / (_SCALE * _KNN + 1e-8))
    w3 = w.reshape(r, _KNN, _HEADS)
    m = jnp.max(w3, axis=1, keepdims=True)
    e = jnp.exp(w3 - m)
    a3 = e / jnp.sum(e, axis=1, keepdims=True)
    a = a3.reshape(g, _HEADS)
    hexp = (lax.broadcasted_iota(jnp.int32, (_HEADS, _DIM), 0)
            == lax.broadcasted_iota(jnp.int32, (_HEADS, _DIM), 1) // _DH).astype(f32)
    a_exp = jnp.dot(a, hexp, preferred_element_type=f32)
    texp = (lax.broadcasted_iota(jnp.int32, (_DH, _DIM), 0)
            == lax.broadcasted_iota(jnp.int32, (_DH, _DIM), 1) % _DH).astype(f32)
    pe_exp = jnp.dot(pe, texp, preferred_element_type=f32)
    contrib = (vg + pe_exp) * a_exp
    out_ref[:, :] = jnp.sum(contrib.reshape(r, _KNN, _DIM), axis=1)


_KVP = 2 * _DIM + 128


def _gather_body(idx_hbm, kvp_hbm, out_hbm, idx_v, buf, sem, *, total):
    c = lax.axis_index("c")
    s = lax.axis_index("s")
    wid = s * _NC + c
    per_w = total // _NW
    base = wid * per_w

    def chunk(t, carry):
        row0 = base + t * _CH
        pltpu.sync_copy(idx_hbm.at[pl.ds(row0, _CH)], idx_v)
        pltpu.async_copy(kvp_hbm.at[idx_v], buf, sem).wait()
        pltpu.sync_copy(buf, out_hbm.at[pl.ds(row0, _CH)])
        return carry

    lax.fori_loop(0, per_w // _CH, chunk, 0)


def _knn_call(n, lo_blk, hi_blk, pqx, pqy, pqz, bq2, pkx, pky, pkz, bkr):
    nblk = n // _R
    nrow = pkx.shape[0]
    grid_spec = pltpu.PrefetchScalarGridSpec(
        num_scalar_prefetch=2,
        grid=(nblk,),
        in_specs=[
            pl.BlockSpec((_R, 1), lambda i, lo, hi: (i, 0)),
            pl.BlockSpec((_R, 1), lambda i, lo, hi: (i, 0)),
            pl.BlockSpec((_R, 1), lambda i, lo, hi: (i, 0)),
            pl.BlockSpec((_R, 1), lambda i, lo, hi: (i, 0)),
            pl.BlockSpec((nrow, _C), lambda i, lo, hi: (0, 0)),
            pl.BlockSpec((nrow, _C), lambda i, lo, hi: (0, 0)),
            pl.BlockSpec((nrow, _C), lambda i, lo, hi: (0, 0)),
            pl.BlockSpec((nrow, _C), lambda i, lo, hi: (0, 0)),
        ],
        out_specs=pl.BlockSpec((_R, _KNN), lambda i, lo, hi: (i, 0)),
    )
    return pl.pallas_call(
        _knn_body,
        grid_spec=grid_spec,
        out_shape=jax.ShapeDtypeStruct((n, _KNN), jnp.int32),
    )(lo_blk, hi_blk, pqx, pqy, pqz, bq2, pkx, pky, pkz, bkr)


def _attn_call(n, Q, kvpg, ppad, w1pad, b1pad, w2pad, b2row):
    nblk = n // _R
    return pl.pallas_call(
        _attn_body,
        grid=(nblk,),
        in_specs=[
            pl.BlockSpec((_R, _DIM), lambda i: (i, 0)),
            pl.BlockSpec((_R * _KNN, _KVP), lambda i: (i, 0)),
            pl.BlockSpec((_R, 16), lambda i: (i, 0)),
            pl.BlockSpec((16, 16), lambda i: (0, 0)),
            pl.BlockSpec((1, 16), lambda i: (0, 0)),
            pl.BlockSpec((16, _DH), lambda i: (0, 0)),
            pl.BlockSpec((1, _DH), lambda i: (0, 0)),
        ],
        out_specs=pl.BlockSpec((_R, _DIM), lambda i: (i, 0)),
        out_shape=jax.ShapeDtypeStruct((n, _DIM), jnp.float32),
    )(Q, kvpg, ppad, w1pad, b1pad, w2pad, b2row)


def _gather_call(n, idx_flat, kvp):
    total = n * _KNN
    mesh = plsc.VectorSubcoreMesh(core_axis_name="c", subcore_axis_name="s")
    body = functools.partial(_gather_body, total=total)
    return pl.kernel(
        body,
        out_type=jax.ShapeDtypeStruct((total, _KVP), jnp.float32),
        mesh=mesh,
        scratch_types=[
            pltpu.VMEM((_CH,), jnp.int32),
            pltpu.VMEM((_CH, _KVP), jnp.float32),
            pltpu.SemaphoreType.DMA,
        ],
    )(idx_flat, kvp)


def kernel(p, Q, K, V, b, W1, b1, gamma, beta, rm, rv, W2, b2):
    n = Q.shape[0]
    f32 = jnp.float32
    sc = gamma / jnp.sqrt(rv + 1e-5)
    w1f = W1 * sc[None, :]
    b1f = (b1 - rm) * sc + beta
    w1pad = jnp.zeros((16, 16), f32).at[:3, :3].set(w1f)
    b1pad = jnp.zeros((1, 16), f32).at[0, :3].set(b1f)
    w2pad = jnp.zeros((16, _DH), f32).at[:3, :].set(W2)
    b2row = b2.reshape(1, _DH).astype(f32)

    offs = jnp.searchsorted(b, jnp.arange(_NB + 1, dtype=b.dtype)).astype(jnp.int32)
    first = b[::_R]
    last = b[_R - 1::_R]
    lo_blk = offs[first]
    hi_blk = offs[last + 1]

    npad = ((n + _C - 1) // _C) * _C
    padn = npad - n
    px = jnp.pad(p[:, 0], (0, padn)).reshape(npad // _C, _C)
    py = jnp.pad(p[:, 1], (0, padn)).reshape(npad // _C, _C)
    pz = jnp.pad(p[:, 2], (0, padn)).reshape(npad // _C, _C)
    bkr = jnp.pad(b, (0, padn), constant_values=-1).reshape(npad // _C, _C)
    pqx = p[:, 0].reshape(n, 1)
    pqy = p[:, 1].reshape(n, 1)
    pqz = p[:, 2].reshape(n, 1)
    bq2 = b.reshape(n, 1)

    ppad = jnp.zeros((n, 16), f32).at[:, :3].set(p)
    kvp = jnp.concatenate(
        [K, V, ppad, jnp.zeros((n, _KVP - 2 * _DIM - 16), f32)], axis=1)

    half = n // 2
    hb = half // _R
    outs = []
    for h in range(2):
        rows = slice(h * half, (h + 1) * half)
        blks = slice(h * hb, (h + 1) * hb)
        idx = _knn_call(half, lo_blk[blks], hi_blk[blks], pqx[rows], pqy[rows],
                        pqz[rows], bq2[rows], px, py, pz, bkr)
        kvpg = _gather_call(half, idx.reshape(half * _KNN), kvp)
        outs.append(_attn_call(half, Q[rows], kvpg, ppad[rows],
                               w1pad, b1pad, w2pad, b2row))
    return jnp.concatenate(outs, axis=0)


# speedup vs baseline: 12.0599x; 1.0340x over previous
import functools

import jax
import jax.numpy as jnp
from jax import lax
from jax.experimental import pallas as pl
from jax.experimental.pallas import tpu as pltpu
from jax.experimental.pallas import tpu_sc as plsc

_HEADS = 8
_DIM = 256
_KNN = 16
_SCALE = 5.656854249
_NB = 4
_DH = _DIM // _HEADS

_R = 128
_C = 1024
_NC = 2
_NS = 16
_NW = _NC * _NS
_CH = 128


def _knn_body(lo_ref, hi_ref, pqx_ref, pqy_ref, pqz_ref, bq_ref,
              pkx_ref, pky_ref, pkz_ref, bk_ref, out_ref):
    i = pl.program_id(0)
    lo = lo_ref[i]
    hi = hi_ref[i]
    qx = pqx_ref[:, :]
    qy = pqy_ref[:, :]
    qz = pqz_ref[:, :]
    bq = bq_ref[:, :]
    r = qx.shape[0]
    cw = _C + _KNN
    lane = lax.broadcasted_iota(jnp.int32, (r, cw), 1)
    lane_k = lax.broadcasted_iota(jnp.int32, (r, _KNN), 1)
    inf = jnp.float32(jnp.inf)

    def chunk(t, carry):
        vals, gidx = carry
        kx = pkx_ref[pl.ds(t, 1), :]
        ky = pky_ref[pl.ds(t, 1), :]
        kz = pkz_ref[pl.ds(t, 1), :]
        bk = bk_ref[pl.ds(t, 1), :]
        d = (qx - kx) ** 2 + (qy - ky) ** 2 + (qz - kz) ** 2
        d = jnp.where(bq == bk, d, inf)
        work = jnp.concatenate([d, vals], axis=1)
        s = t * _C
        new_vals = []
        new_idx = []
        for _ in range(_KNN):
            mn = jnp.min(work, axis=1, keepdims=True)
            le = work <= mn
            col = jnp.min(jnp.where(le, lane, cw), axis=1, keepdims=True)
            carry_pick = jnp.sum(
                jnp.where(lane_k == col - _C, gidx, 0), axis=1, keepdims=True)
            pick = jnp.where(col < _C, s + col, carry_pick)
            new_vals.append(mn)
            new_idx.append(pick)
            work = jnp.where(le, inf, work)
        return jnp.concatenate(new_vals, axis=1), jnp.concatenate(new_idx, axis=1)

    vals0 = jnp.full((r, _KNN), inf, dtype=jnp.float32)
    gidx0 = jnp.zeros((r, _KNN), dtype=jnp.int32)
    _, gidx = lax.fori_loop(lo // _C, (hi + _C - 1) // _C, chunk, (vals0, gidx0))
    out_ref[:, :] = gidx


def _attn_body(q_ref, kvp_ref, pq_ref,
               w1_ref, b1_ref, w2_ref, b2_ref, out_ref):
    f32 = jnp.float32
    r = q_ref.shape[0]
    g = r * _KNN
    q = q_ref[:, :]
    kg = kvp_ref[:, :_DIM]
    vg = kvp_ref[:, _DIM:2 * _DIM]
    pg = kvp_ref[:, 2 * _DIM:2 * _DIM + 16]
    pq = pq_ref[:, :]
    q_rep = jnp.broadcast_to(q[:, None, :], (r, _KNN, _DIM)).reshape(g, _DIM)
    pq_rep = jnp.broadcast_to(pq[:, None, :], (r, _KNN, 16)).reshape(g, 16)
    dp = pq_rep - pg
    h = jnp.dot(dp, w1_ref[:, :], preferred_element_type=f32) + b1_ref[:, :]
    h = jnp.maximum(h, 0.0)
    pe = jnp.dot(h, w2_ref[:, :], preferred_element_type=f32) + b2_ref[:, :]
    s_pe = jnp.sum(pe, axis=1, keepdims=True)
    hsel = (lax.broadcasted_iota(jnp.int32, (_DIM, _HEADS), 0) // _DH
            == lax.broadcasted_iota(jnp.int32, (_DIM, _HEADS), 1)).astype(f32)
    qk = jnp.dot(kg * q_rep, hsel, preferred_element_type=f32)
    w = (qk + s_pe) * (1.0 / (_SCALE * _KNN + 1e-8))
    w3 = w.reshape(r, _KNN, _HEADS)
    m = jnp.max(w3, axis=1, keepdims=True)
    e = jnp.exp(w3 - m)
    a3 = e / jnp.sum(e, axis=1, keepdims=True)
    a = a3.reshape(g, _HEADS)
    hexp = (lax.broadcasted_iota(jnp.int32, (_HEADS, _DIM), 0)
            == lax.broadcasted_iota(jnp.int32, (_HEADS, _DIM), 1) // _DH).astype(f32)
    a_exp = jnp.dot(a, hexp, preferred_element_type=f32)
    texp = (lax.broadcasted_iota(jnp.int32, (_DH, _DIM), 0)
            == lax.broadcasted_iota(jnp.int32, (_DH, _DIM), 1) % _DH).astype(f32)
    pe_exp = jnp.dot(pe, texp, preferred_element_type=f32)
    contrib = (vg + pe_exp) * a_exp
    out_ref[:, :] = jnp.sum(contrib.reshape(r, _KNN, _DIM), axis=1)


_KVP = 2 * _DIM + 128


def _gather_body(idx_hbm, kvp_hbm, out_hbm, idx_v, buf, sem, *, total):
    c = lax.axis_index("c")
    s = lax.axis_index("s")
    wid = s * _NC + c
    per_w = total // _NW
    base = wid * per_w

    def chunk(t, carry):
        row0 = base + t * _CH
        pltpu.sync_copy(idx_hbm.at[pl.ds(row0, _CH)], idx_v)
        pltpu.async_copy(kvp_hbm.at[idx_v], buf, sem).wait()
        pltpu.sync_copy(buf, out_hbm.at[pl.ds(row0, _CH)])
        return carry

    lax.fori_loop(0, per_w // _CH, chunk, 0)


def _knn_call(n, lo_blk, hi_blk, pqx, pqy, pqz, bq2, pkx, pky, pkz, bkr):
    nblk = n // _R
    nrow = pkx.shape[0]
    grid_spec = pltpu.PrefetchScalarGridSpec(
        num_scalar_prefetch=2,
        grid=(nblk,),
        in_specs=[
            pl.BlockSpec((_R, 1), lambda i, lo, hi: (i, 0)),
            pl.BlockSpec((_R, 1), lambda i, lo, hi: (i, 0)),
            pl.BlockSpec((_R, 1), lambda i, lo, hi: (i, 0)),
            pl.BlockSpec((_R, 1), lambda i, lo, hi: (i, 0)),
            pl.BlockSpec((nrow, _C), lambda i, lo, hi: (0, 0)),
            pl.BlockSpec((nrow, _C), lambda i, lo, hi: (0, 0)),
            pl.BlockSpec((nrow, _C), lambda i, lo, hi: (0, 0)),
            pl.BlockSpec((nrow, _C), lambda i, lo, hi: (0, 0)),
        ],
        out_specs=pl.BlockSpec((_R, _KNN), lambda i, lo, hi: (i, 0)),
    )
    return pl.pallas_call(
        _knn_body,
        grid_spec=grid_spec,
        out_shape=jax.ShapeDtypeStruct((n, _KNN), jnp.int32),
    )(lo_blk, hi_blk, pqx, pqy, pqz, bq2, pkx, pky, pkz, bkr)


def _attn_call(n, Q, kvpg, ppad, w1pad, b1pad, w2pad, b2row):
    nblk = n // _R
    return pl.pallas_call(
        _attn_body,
        grid=(nblk,),
        in_specs=[
            pl.BlockSpec((_R, _DIM), lambda i: (i, 0)),
            pl.BlockSpec((_R * _KNN, _KVP), lambda i: (i, 0)),
            pl.BlockSpec((_R, 16), lambda i: (i, 0)),
            pl.BlockSpec((16, 16), lambda i: (0, 0)),
            pl.BlockSpec((1, 16), lambda i: (0, 0)),
            pl.BlockSpec((16, _DH), lambda i: (0, 0)),
            pl.BlockSpec((1, _DH), lambda i: (0, 0)),
        ],
        out_specs=pl.BlockSpec((_R, _DIM), lambda i: (i, 0)),
        out_shape=jax.ShapeDtypeStruct((n, _DIM), jnp.float32),
    )(Q, kvpg, ppad, w1pad, b1pad, w2pad, b2row)


def _gather_call(n, idx_flat, kvp):
    total = n * _KNN
    mesh = plsc.VectorSubcoreMesh(core_axis_name="c", subcore_axis_name="s")
    body = functools.partial(_gather_body, total=total)
    return pl.kernel(
        body,
        out_type=jax.ShapeDtypeStruct((total, _KVP), jnp.float32),
        mesh=mesh,
        scratch_types=[
            pltpu.VMEM((_CH,), jnp.int32),
            pltpu.VMEM((_CH, _KVP), jnp.float32),
            pltpu.SemaphoreType.DMA,
        ],
    )(idx_flat, kvp)


def kernel(p, Q, K, V, b, W1, b1, gamma, beta, rm, rv, W2, b2):
    n = Q.shape[0]
    f32 = jnp.float32
    sc = gamma / jnp.sqrt(rv + 1e-5)
    w1f = W1 * sc[None, :]
    b1f = (b1 - rm) * sc + beta
    w1pad = jnp.zeros((16, 16), f32).at[:3, :3].set(w1f)
    b1pad = jnp.zeros((1, 16), f32).at[0, :3].set(b1f)
    w2pad = jnp.zeros((16, _DH), f32).at[:3, :].set(W2)
    b2row = b2.reshape(1, _DH).astype(f32)

    offs = jnp.searchsorted(b, jnp.arange(_NB + 1, dtype=b.dtype)).astype(jnp.int32)
    first = b[::_R]
    last = b[_R - 1::_R]
    lo_blk = offs[first]
    hi_blk = offs[last + 1]

    npad = ((n + _C - 1) // _C) * _C
    padn = npad - n
    px = jnp.pad(p[:, 0], (0, padn)).reshape(npad // _C, _C)
    py = jnp.pad(p[:, 1], (0, padn)).reshape(npad // _C, _C)
    pz = jnp.pad(p[:, 2], (0, padn)).reshape(npad // _C, _C)
    bkr = jnp.pad(b, (0, padn), constant_values=-1).reshape(npad // _C, _C)
    pqx = p[:, 0].reshape(n, 1)
    pqy = p[:, 1].reshape(n, 1)
    pqz = p[:, 2].reshape(n, 1)
    bq2 = b.reshape(n, 1)

    ppad = jnp.zeros((n, 16), f32).at[:, :3].set(p)
    kvp = jnp.concatenate(
        [K, V, ppad, jnp.zeros((n, _KVP - 2 * _DIM - 16), f32)], axis=1)

    npipe = 4
    half = n // npipe
    hb = half // _R
    outs = []
    for h in range(npipe):
        rows = slice(h * half, (h + 1) * half)
        blks = slice(h * hb, (h + 1) * hb)
        idx = _knn_call(half, lo_blk[blks], hi_blk[blks], pqx[rows], pqy[rows],
                        pqz[rows], bq2[rows], px, py, pz, bkr)
        kvpg = _gather_call(half, idx.reshape(half * _KNN), kvp)
        outs.append(_attn_call(half, Q[rows], kvpg, ppad[rows],
                               w1pad, b1pad, w2pad, b2row))
    return jnp.concatenate(outs, axis=0)
